# baseline (speedup 1.0000x reference)
import jax
import jax.numpy as jnp
from jax.experimental import pallas as pl
from jax.experimental.pallas import tpu as pltpu

_BT = 512


def _fused_body(s_ref, n_ref, W1_ref, b1_ref, W2_ref, b2_ref,
                W3_ref, b3_ref, W4_ref, b4_ref, out_ref):
    bt = s_ref.shape[0]
    x = jnp.concatenate([s_ref[...], n_ref[...]], axis=0)
    h = jnp.dot(x, W1_ref[...], preferred_element_type=jnp.float32)
    h = jnp.maximum(h + b1_ref[...], 0.0)
    y = jnp.dot(h, W2_ref[...], preferred_element_type=jnp.float32)
    y = jnp.maximum(y + b2_ref[...], 0.0)
    y2 = jnp.concatenate([y[:bt], y[bt:]], axis=1)
    h3 = jnp.dot(y2, W3_ref[...], preferred_element_type=jnp.float32)
    h3 = jnp.maximum(h3 + b3_ref[...], 0.0)
    out = jnp.dot(h3, W4_ref[...], preferred_element_type=jnp.float32)
    out_ref[...] = out + b4_ref[...]


def kernel(state, next_state, W1, b1, W2, b2, W3, b3, W4, b4):
    B, sd = state.shape
    mid = W1.shape[1]
    out_dim = W4.shape[1]
    grid = (B // _BT,)

    def _tile(i):
        return (i, 0)

    def _whole(i):
        return (0, 0)

    full = lambda a: pl.BlockSpec(a.shape, _whole)
    b1r, b2r, b3r, b4r = (b.reshape(1, -1) for b in (b1, b2, b3, b4))

    return pl.pallas_call(
        _fused_body,
        grid=grid,
        in_specs=[
            pl.BlockSpec((_BT, sd), _tile),
            pl.BlockSpec((_BT, sd), _tile),
            full(W1), full(b1r), full(W2), full(b2r),
            full(W3), full(b3r), full(W4), full(b4r),
        ],
        out_specs=pl.BlockSpec((_BT, out_dim), _tile),
        out_shape=jax.ShapeDtypeStruct((B, out_dim), jnp.float32),
        compiler_params=pltpu.CompilerParams(
            dimension_semantics=("arbitrary",),
            vmem_limit_bytes=100 * 1024 * 1024,
        ),
    )(state, next_state, W1, b1r, W2, b2r, W3, b3r, W4, b4r)
